# Initial kernel scaffold; baseline (speedup 1.0000x reference)
import jax
import jax.numpy as jnp
from jax.experimental import pallas as pl


def kernel(x, target, mel_max_length, conv1_w, conv1_b, ln1_g, ln1_b, conv2_w, conv2_b, ln2_g, ln2_b, lin_w, lin_b):
    raise NotImplementedError("write your pallas kernel here")


# speedup vs baseline: 83.1945x; 83.1945x over previous
import functools

import jax
import jax.numpy as jnp
from jax import lax
from jax.experimental import pallas as pl
from jax.experimental.pallas import tpu as pltpu
from jax.experimental.pallas import tpu_sc as plsc

B, L, C, M = 16, 512, 256, 4096
NC, NS = 2, 16
NW = NC * NS
FPW = (B * M) // NW
CH = 128
NCHK = FPW // CH
LANES = 16


def _sc_body(x_hbm, tgt_hbm, out_hbm, tgt_v, arr_v, gidx_v, buf, zbuf, sem):
    cid = lax.axis_index("c")
    sid = lax.axis_index("s")
    wid = sid * NC + cid
    b = wid // 2
    t0 = (wid % 2) * FPW

    pltpu.sync_copy(tgt_hbm.at[b], tgt_v)

    izeros = jnp.zeros((LANES,), jnp.int32)
    fzeros = jnp.zeros((LANES,), jnp.float32)
    lane = lax.iota(jnp.int32, LANES)

    def _zero_arr(i, c):
        arr_v[pl.ds(i * LANES, LANES)] = izeros
        return c

    lax.fori_loop(0, FPW // LANES, _zero_arr, 0)

    def _zero_zbuf(r, c):
        for cc in range(C // LANES):
            zbuf[r, pl.ds(cc * LANES, LANES)] = fzeros
        return c

    lax.fori_loop(0, CH, _zero_zbuf, 0)

    def _pass1(j, carry):
        run, cnt = carry
        d = tgt_v[pl.ds(j * LANES, LANES)]
        c = plsc.cumsum(d) + run
        start = c - d
        jid = lane + j * LANES
        m = (d > 0) & (start >= t0) & (start < t0 + FPW)
        plsc.store_scatter(arr_v, [start - t0], jid, mask=m)
        run = jnp.max(c)
        cnt = cnt + jnp.sum((c <= t0).astype(jnp.int32))
        return run, cnt

    total, carry0 = lax.fori_loop(
        0, L // LANES, _pass1, (jnp.int32(0), jnp.int32(0)))

    def _pass2(i, carry):
        a = arr_v[pl.ds(i * LANES, LANES)]
        mval = jnp.maximum(plsc.cummax(a), carry)
        row = b * L + jnp.minimum(mval, L - 1)
        gidx_v[pl.ds(i * LANES, LANES)] = row
        return jnp.max(mval)

    lax.fori_loop(0, FPW // LANES, _pass2, carry0)

    valid = total - t0

    for k in range(NCHK):
        vs = valid - k * CH
        obase = b * M + t0 + k * CH

        @pl.when(vs > 0)
        def _gather_chunk():
            pltpu.async_copy(
                x_hbm.at[gidx_v.at[pl.ds(k * CH, CH)]], buf, sem).wait()

            def _zero_tail(r, c):
                for cc in range(C // LANES):
                    buf[r, pl.ds(cc * LANES, LANES)] = fzeros
                return c

            lax.fori_loop(jnp.clip(vs, 0, CH), CH, _zero_tail, 0)
            pltpu.sync_copy(buf, out_hbm.at[pl.ds(obase, CH)])

        @pl.when(vs <= 0)
        def _zero_chunk():
            pltpu.sync_copy(zbuf, out_hbm.at[pl.ds(obase, CH)])


def _sc_expand(x2d, tgt):
    mesh = plsc.VectorSubcoreMesh(core_axis_name="c", subcore_axis_name="s")
    kern = pl.kernel(
        _sc_body,
        out_type=jax.ShapeDtypeStruct((B * M, C), jnp.float32),
        mesh=mesh,
        scratch_types=[
            pltpu.VMEM((L,), jnp.int32),
            pltpu.VMEM((FPW,), jnp.int32),
            pltpu.VMEM((FPW,), jnp.int32),
            pltpu.VMEM((CH, C), jnp.float32),
            pltpu.VMEM((CH, C), jnp.float32),
            pltpu.SemaphoreType.DMA,
        ],
        compiler_params=pltpu.CompilerParams(needs_layout_passes=False),
    )
    return kern(x2d, tgt)


def _layernorm(h, g, bb):
    mu = jnp.mean(h, axis=-1, keepdims=True)
    var = jnp.mean((h - mu) ** 2, axis=-1, keepdims=True)
    return (h - mu) * lax.rsqrt(var + 1e-5) * g + bb


def _conv_block(X, w0, w1, w2, bias):
    z = jnp.zeros((1, C), jnp.float32)
    Xm = jnp.concatenate([z, X[:-1]], axis=0)
    Xp = jnp.concatenate([X[1:], z], axis=0)
    f32 = jnp.float32
    h = (jnp.dot(Xm, w0, preferred_element_type=f32)
         + jnp.dot(X, w1, preferred_element_type=f32)
         + jnp.dot(Xp, w2, preferred_element_type=f32))
    return h + bias


def _vp_body(x_ref, w10, w11, w12, b1, g1, e1, w20, w21, w22, b2, g2, e2,
             lw, lb, out_ref):
    X = x_ref[0]
    h = _conv_block(X, w10[:], w11[:], w12[:], b1[:])
    h = _layernorm(jnp.maximum(h, 0.0), g1[:], e1[:])
    h = _conv_block(h, w20[:], w21[:], w22[:], b2[:])
    h = _layernorm(jnp.maximum(h, 0.0), g2[:], e2[:])
    o = jnp.dot(h, lw[:], preferred_element_type=jnp.float32) + lb[:]
    out_ref[0] = jnp.maximum(o, 0.0)


def _variance_predictor(x, conv1_w, conv1_b, ln1_g, ln1_b, conv2_w, conv2_b,
                        ln2_g, ln2_b, lin_w, lin_b):
    full2d = pl.BlockSpec((C, C), lambda i: (0, 0))
    vec = pl.BlockSpec((C,), lambda i: (0,))
    out = pl.pallas_call(
        _vp_body,
        grid=(B,),
        in_specs=[
            pl.BlockSpec((1, L, C), lambda i: (i, 0, 0)),
            full2d, full2d, full2d, vec, vec, vec,
            full2d, full2d, full2d, vec, vec, vec,
            pl.BlockSpec((C, 1), lambda i: (0, 0)),
            pl.BlockSpec((1,), lambda i: (0,)),
        ],
        out_specs=pl.BlockSpec((1, L, 1), lambda i: (i, 0, 0)),
        out_shape=jax.ShapeDtypeStruct((B, L, 1), jnp.float32),
    )(
        x,
        conv1_w[:, :, 0].T, conv1_w[:, :, 1].T, conv1_w[:, :, 2].T,
        conv1_b, ln1_g, ln1_b,
        conv2_w[:, :, 0].T, conv2_w[:, :, 1].T, conv2_w[:, :, 2].T,
        conv2_b, ln2_g, ln2_b,
        lin_w.T, lin_b,
    )
    return out.reshape(B, L)


def kernel(x, target, mel_max_length, conv1_w, conv1_b, ln1_g, ln1_b,
           conv2_w, conv2_b, ln2_g, ln2_b, lin_w, lin_b):
    ldp = _variance_predictor(x, conv1_w, conv1_b, ln1_g, ln1_b,
                              conv2_w, conv2_b, ln2_g, ln2_b, lin_w, lin_b)
    out = _sc_expand(x.reshape(B * L, C), target).reshape(B, M, C)
    return (out, ldp)


# speedup vs baseline: 89.0572x; 1.0705x over previous
import functools

import jax
import jax.numpy as jnp
from jax import lax
from jax.experimental import pallas as pl
from jax.experimental.pallas import tpu as pltpu
from jax.experimental.pallas import tpu_sc as plsc

B, L, C, M = 16, 512, 256, 4096
NC, NS = 2, 16
NW = NC * NS
FPW = (B * M) // NW
CH = 128
CPW = (M // CH) // NC
LANES = 16


def _sc_body(x_hbm, tgt_hbm, out_hbm, tgt_v, arr_v, gidx_v,
             buf_a, buf_b, zbuf, gsem_a, gsem_b, wsem_a, wsem_b):
    cid = lax.axis_index("c")
    sid = lax.axis_index("s")
    b = sid
    par = cid

    pltpu.sync_copy(tgt_hbm.at[b], tgt_v)

    izeros = jnp.zeros((LANES,), jnp.int32)
    fzeros = jnp.zeros((LANES,), jnp.float32)
    lane = lax.iota(jnp.int32, LANES)

    def _zero_arr(i, c):
        arr_v[pl.ds(i * LANES, LANES)] = izeros
        return c

    lax.fori_loop(0, M // LANES, _zero_arr, 0)

    def _zero_zbuf(r, c):
        for cc in range(C // LANES):
            zbuf[r, pl.ds(cc * LANES, LANES)] = fzeros
        return c

    lax.fori_loop(0, CH, _zero_zbuf, 0)

    def _pass1(j, carry):
        run, cnt = carry
        d = tgt_v[pl.ds(j * LANES, LANES)]
        c = plsc.cumsum(d) + run
        start = c - d
        jid = lane + j * LANES
        m = (d > 0) & (start < M)
        plsc.store_scatter(arr_v, [start], jid, mask=m)
        run = jnp.max(c)
        cnt = cnt + jnp.sum((c <= 0).astype(jnp.int32))
        return run, cnt

    total, carry0 = lax.fori_loop(
        0, L // LANES, _pass1, (jnp.int32(0), jnp.int32(0)))

    def _pass2(i, carry):
        a = arr_v[pl.ds(i * LANES, LANES)]
        mval = jnp.maximum(plsc.cummax(a), carry)
        row = b * L + jnp.minimum(mval, L - 1)
        gidx_v[pl.ds(i * LANES, LANES)] = row
        return jnp.max(mval)

    lax.fori_loop(0, M // LANES, _pass2, carry0)

    valid = total

    bufs = (buf_a, buf_b)
    gsems = (gsem_a, gsem_b)
    wsems = (wsem_a, wsem_b)
    chunks = [2 * i + par for i in range(CPW)]

    def _start_gather(k, p):
        pltpu.async_copy(
            x_hbm.at[gidx_v.at[pl.ds(k * CH, CH)]], bufs[p], gsems[p])

    def _wait_gather(k, p):
        pltpu.make_async_copy(
            x_hbm.at[gidx_v.at[pl.ds(k * CH, CH)]], bufs[p], gsems[p]).wait()

    def _obase(k):
        return b * M + k * CH

    def _wait_write(k, p):
        pltpu.make_async_copy(
            zbuf, out_hbm.at[pl.ds(_obase(k), CH)], wsems[p]).wait()

    for i, k in enumerate(chunks):
        vs = valid - k * CH
        p = i % 2

        if i == 0:
            @pl.when(vs > 0)
            def _prime():
                _start_gather(k, 0)

        @pl.when(vs > 0)
        def _data_chunk():
            _wait_gather(k, p)

            def _zero_tail(r, c):
                for cc in range(C // LANES):
                    bufs[p][r, pl.ds(cc * LANES, LANES)] = fzeros
                return c

            lax.fori_loop(jnp.clip(vs, 0, CH), CH, _zero_tail, 0)
            pltpu.async_copy(bufs[p], out_hbm.at[pl.ds(_obase(k), CH)],
                             wsems[p])

        @pl.when(vs <= 0)
        def _zero_chunk():
            pltpu.async_copy(zbuf, out_hbm.at[pl.ds(_obase(k), CH)],
                             wsems[p])

        if i + 1 < CPW:
            kn = chunks[i + 1]
            vs_n = valid - kn * CH
            q = (i + 1) % 2
            if i >= 1:
                _wait_write(chunks[i - 1], q)

            @pl.when(vs_n > 0)
            def _next_gather():
                _start_gather(kn, q)

    _wait_write(chunks[CPW - 1], (CPW - 1) % 2)
    _wait_write(chunks[CPW - 2], (CPW - 2) % 2)


def _sc_expand(x2d, tgt):
    mesh = plsc.VectorSubcoreMesh(core_axis_name="c", subcore_axis_name="s")
    kern = pl.kernel(
        _sc_body,
        out_type=jax.ShapeDtypeStruct((B * M, C), jnp.float32),
        mesh=mesh,
        scratch_types=[
            pltpu.VMEM((L,), jnp.int32),
            pltpu.VMEM((M,), jnp.int32),
            pltpu.VMEM((M,), jnp.int32),
            pltpu.VMEM((CH, C), jnp.float32),
            pltpu.VMEM((CH, C), jnp.float32),
            pltpu.VMEM((CH, C), jnp.float32),
            pltpu.SemaphoreType.DMA,
            pltpu.SemaphoreType.DMA,
            pltpu.SemaphoreType.DMA,
            pltpu.SemaphoreType.DMA,
        ],
        compiler_params=pltpu.CompilerParams(needs_layout_passes=False),
    )
    return kern(x2d, tgt)


def _layernorm(h, g, bb):
    mu = jnp.mean(h, axis=-1, keepdims=True)
    var = jnp.mean((h - mu) ** 2, axis=-1, keepdims=True)
    return (h - mu) * lax.rsqrt(var + 1e-5) * g + bb


def _conv_block(X, w0, w1, w2, bias):
    z = jnp.zeros((1, C), jnp.float32)
    Xm = jnp.concatenate([z, X[:-1]], axis=0)
    Xp = jnp.concatenate([X[1:], z], axis=0)
    f32 = jnp.float32
    h = (jnp.dot(Xm, w0, preferred_element_type=f32)
         + jnp.dot(X, w1, preferred_element_type=f32)
         + jnp.dot(Xp, w2, preferred_element_type=f32))
    return h + bias


def _vp_body(x_ref, w10, w11, w12, b1, g1, e1, w20, w21, w22, b2, g2, e2,
             lw, lb, out_ref):
    X = x_ref[0]
    h = _conv_block(X, w10[:], w11[:], w12[:], b1[:])
    h = _layernorm(jnp.maximum(h, 0.0), g1[:], e1[:])
    h = _conv_block(h, w20[:], w21[:], w22[:], b2[:])
    h = _layernorm(jnp.maximum(h, 0.0), g2[:], e2[:])
    o = jnp.dot(h, lw[:], preferred_element_type=jnp.float32) + lb[:]
    out_ref[0] = jnp.maximum(o, 0.0)


def _variance_predictor(x, conv1_w, conv1_b, ln1_g, ln1_b, conv2_w, conv2_b,
                        ln2_g, ln2_b, lin_w, lin_b):
    full2d = pl.BlockSpec((C, C), lambda i: (0, 0))
    vec = pl.BlockSpec((C,), lambda i: (0,))
    out = pl.pallas_call(
        _vp_body,
        grid=(B,),
        in_specs=[
            pl.BlockSpec((1, L, C), lambda i: (i, 0, 0)),
            full2d, full2d, full2d, vec, vec, vec,
            full2d, full2d, full2d, vec, vec, vec,
            pl.BlockSpec((C, 1), lambda i: (0, 0)),
            pl.BlockSpec((1,), lambda i: (0,)),
        ],
        out_specs=pl.BlockSpec((1, L, 1), lambda i: (i, 0, 0)),
        out_shape=jax.ShapeDtypeStruct((B, L, 1), jnp.float32),
    )(
        x,
        conv1_w[:, :, 0].T, conv1_w[:, :, 1].T, conv1_w[:, :, 2].T,
        conv1_b, ln1_g, ln1_b,
        conv2_w[:, :, 0].T, conv2_w[:, :, 1].T, conv2_w[:, :, 2].T,
        conv2_b, ln2_g, ln2_b,
        lin_w.T, lin_b,
    )
    return out.reshape(B, L)


def kernel(x, target, mel_max_length, conv1_w, conv1_b, ln1_g, ln1_b,
           conv2_w, conv2_b, ln2_g, ln2_b, lin_w, lin_b):
    ldp = _variance_predictor(x, conv1_w, conv1_b, ln1_g, ln1_b,
                              conv2_w, conv2_b, ln2_g, ln2_b, lin_w, lin_b)
    out = _sc_expand(x.reshape(B * L, C), target).reshape(B, M, C)
    return (out, ldp)


# speedup vs baseline: 89.5542x; 1.0056x over previous
import functools

import jax
import jax.numpy as jnp
from jax import lax
from jax.experimental import pallas as pl
from jax.experimental.pallas import tpu as pltpu
from jax.experimental.pallas import tpu_sc as plsc

B, L, C, M = 16, 512, 256, 4096
NC, NS = 2, 16
NW = NC * NS
FPW = (B * M) // NW
CH = 128
CPW = (M // CH) // NC
LANES = 16


def _sc_body(x_hbm, tgt_hbm, out_hbm, tgt_v, arr_v, gidx_v,
             buf_a, buf_b, zbuf, gsem_a, gsem_b, wsem_a, wsem_b):
    cid = lax.axis_index("c")
    sid = lax.axis_index("s")
    b = sid
    par = cid

    pltpu.sync_copy(tgt_hbm.at[b], tgt_v)

    izeros = jnp.zeros((LANES,), jnp.int32)
    fzeros = jnp.zeros((LANES,), jnp.float32)
    lane = lax.iota(jnp.int32, LANES)

    with jax.named_scope("zerofill"):
        def _zero_arr(i, c):
            arr_v[pl.ds(i * LANES, LANES)] = izeros
            return c

        lax.fori_loop(0, M // LANES, _zero_arr, 0)

        def _zero_zbuf(r, c):
            for cc in range(C // LANES):
                zbuf[r, pl.ds(cc * LANES, LANES)] = fzeros
            return c

        lax.fori_loop(0, CH, _zero_zbuf, 0)

    def _pass1(j, carry):
        run, cnt = carry
        d = tgt_v[pl.ds(j * LANES, LANES)]
        c = plsc.cumsum(d) + run
        start = c - d
        jid = lane + j * LANES
        m = (d > 0) & (start < M)
        plsc.store_scatter(arr_v, [start], jid, mask=m)
        run = jnp.max(c)
        cnt = cnt + jnp.sum((c <= 0).astype(jnp.int32))
        return run, cnt

    with jax.named_scope("pass1"):
        total, carry0 = lax.fori_loop(
            0, L // LANES, _pass1, (jnp.int32(0), jnp.int32(0)))

    def _pass2(i, carry):
        a = arr_v[pl.ds(i * LANES, LANES)]
        mval = jnp.maximum(plsc.cummax(a), carry)
        row = b * L + jnp.minimum(mval, L - 1)
        gidx_v[pl.ds(i * LANES, LANES)] = row
        return jnp.max(mval)

    with jax.named_scope("pass2"):
        lax.fori_loop(0, M // LANES, _pass2, carry0)

    valid = total

    bufs = (buf_a, buf_b)
    gsems = (gsem_a, gsem_b)
    wsems = (wsem_a, wsem_b)
    chunks = [2 * i + par for i in range(CPW)]

    def _start_gather(k, p):
        pltpu.async_copy(
            x_hbm.at[gidx_v.at[pl.ds(k * CH, CH)]], bufs[p], gsems[p])

    def _wait_gather(k, p):
        pltpu.make_async_copy(
            x_hbm.at[gidx_v.at[pl.ds(k * CH, CH)]], bufs[p], gsems[p]).wait()

    def _obase(k):
        return b * M + k * CH

    def _wait_write(k, p):
        pltpu.make_async_copy(
            zbuf, out_hbm.at[pl.ds(_obase(k), CH)], wsems[p]).wait()

    _chunk_scope = jax.named_scope("chunkloop")
    _chunk_scope.__enter__()
    for i, k in enumerate(chunks):
        vs = valid - k * CH
        p = i % 2

        if i == 0:
            @pl.when(vs > 0)
            def _prime():
                _start_gather(k, 0)

        @pl.when(vs > 0)
        def _data_chunk():
            _wait_gather(k, p)

            def _zero_tail(r, c):
                for cc in range(C // LANES):
                    bufs[p][r, pl.ds(cc * LANES, LANES)] = fzeros
                return c

            lax.fori_loop(jnp.clip(vs, 0, CH), CH, _zero_tail, 0)
            pltpu.async_copy(bufs[p], out_hbm.at[pl.ds(_obase(k), CH)],
                             wsems[p])

        @pl.when(vs <= 0)
        def _zero_chunk():
            pltpu.async_copy(zbuf, out_hbm.at[pl.ds(_obase(k), CH)],
                             wsems[p])

        if i + 1 < CPW:
            kn = chunks[i + 1]
            vs_n = valid - kn * CH
            q = (i + 1) % 2
            if i >= 1:
                _wait_write(chunks[i - 1], q)

            @pl.when(vs_n > 0)
            def _next_gather():
                _start_gather(kn, q)

    _wait_write(chunks[CPW - 1], (CPW - 1) % 2)
    _wait_write(chunks[CPW - 2], (CPW - 2) % 2)
    _chunk_scope.__exit__(None, None, None)


def _sc_expand(x2d, tgt):
    mesh = plsc.VectorSubcoreMesh(core_axis_name="c", subcore_axis_name="s")
    kern = pl.kernel(
        _sc_body,
        out_type=jax.ShapeDtypeStruct((B * M, C), jnp.float32),
        mesh=mesh,
        scratch_types=[
            pltpu.VMEM((L,), jnp.int32),
            pltpu.VMEM((M,), jnp.int32),
            pltpu.VMEM((M,), jnp.int32),
            pltpu.VMEM((CH, C), jnp.float32),
            pltpu.VMEM((CH, C), jnp.float32),
            pltpu.VMEM((CH, C), jnp.float32),
            pltpu.SemaphoreType.DMA,
            pltpu.SemaphoreType.DMA,
            pltpu.SemaphoreType.DMA,
            pltpu.SemaphoreType.DMA,
        ],
        compiler_params=pltpu.CompilerParams(needs_layout_passes=False),
    )
    return kern(x2d, tgt)


def _layernorm(h, g, bb):
    mu = jnp.mean(h, axis=-1, keepdims=True)
    var = jnp.mean((h - mu) ** 2, axis=-1, keepdims=True)
    return (h - mu) * lax.rsqrt(var + 1e-5) * g + bb


def _conv_block(X, w0, w1, w2, bias):
    z = jnp.zeros((1, C), jnp.float32)
    Xm = jnp.concatenate([z, X[:-1]], axis=0)
    Xp = jnp.concatenate([X[1:], z], axis=0)
    f32 = jnp.float32
    h = (jnp.dot(Xm, w0, preferred_element_type=f32)
         + jnp.dot(X, w1, preferred_element_type=f32)
         + jnp.dot(Xp, w2, preferred_element_type=f32))
    return h + bias


def _vp_body(x_ref, w10, w11, w12, b1, g1, e1, w20, w21, w22, b2, g2, e2,
             lw, lb, out_ref):
    X = x_ref[0]
    h = _conv_block(X, w10[:], w11[:], w12[:], b1[:])
    h = _layernorm(jnp.maximum(h, 0.0), g1[:], e1[:])
    h = _conv_block(h, w20[:], w21[:], w22[:], b2[:])
    h = _layernorm(jnp.maximum(h, 0.0), g2[:], e2[:])
    o = jnp.dot(h, lw[:], preferred_element_type=jnp.float32) + lb[:]
    out_ref[0] = jnp.maximum(o, 0.0)


def _variance_predictor(x, conv1_w, conv1_b, ln1_g, ln1_b, conv2_w, conv2_b,
                        ln2_g, ln2_b, lin_w, lin_b):
    full2d = pl.BlockSpec((C, C), lambda i: (0, 0))
    vec = pl.BlockSpec((C,), lambda i: (0,))
    out = pl.pallas_call(
        _vp_body,
        grid=(B,),
        in_specs=[
            pl.BlockSpec((1, L, C), lambda i: (i, 0, 0)),
            full2d, full2d, full2d, vec, vec, vec,
            full2d, full2d, full2d, vec, vec, vec,
            pl.BlockSpec((C, 1), lambda i: (0, 0)),
            pl.BlockSpec((1,), lambda i: (0,)),
        ],
        out_specs=pl.BlockSpec((1, L, 1), lambda i: (i, 0, 0)),
        out_shape=jax.ShapeDtypeStruct((B, L, 1), jnp.float32),
    )(
        x,
        conv1_w[:, :, 0].T, conv1_w[:, :, 1].T, conv1_w[:, :, 2].T,
        conv1_b, ln1_g, ln1_b,
        conv2_w[:, :, 0].T, conv2_w[:, :, 1].T, conv2_w[:, :, 2].T,
        conv2_b, ln2_g, ln2_b,
        lin_w.T, lin_b,
    )
    return out.reshape(B, L)


def kernel(x, target, mel_max_length, conv1_w, conv1_b, ln1_g, ln1_b,
           conv2_w, conv2_b, ln2_g, ln2_b, lin_w, lin_b):
    ldp = _variance_predictor(x, conv1_w, conv1_b, ln1_g, ln1_b,
                              conv2_w, conv2_b, ln2_g, ln2_b, lin_w, lin_b)
    out = _sc_expand(x.reshape(B * L, C), target).reshape(B, M, C)
    return (out, ldp)


# speedup vs baseline: 107.1236x; 1.1962x over previous
import functools

import jax
import jax.numpy as jnp
from jax import lax
from jax.experimental import pallas as pl
from jax.experimental.pallas import tpu as pltpu
from jax.experimental.pallas import tpu_sc as plsc

B, L, C, M = 16, 512, 256, 4096
NC, NS = 2, 16
NW = NC * NS
FPW = (B * M) // NW
CH = 128
CPW = (M // CH) // NC
LANES = 16


def _sc_body(x_hbm, tgt_hbm, out_hbm, tgt_v, arr_v, gidx_v,
             buf_a, buf_b, zbuf, gsem_a, gsem_b, wsem_a, wsem_b, zwsem):
    cid = lax.axis_index("c")
    sid = lax.axis_index("s")
    b = sid
    par = cid

    pltpu.sync_copy(tgt_hbm.at[b], tgt_v)

    izeros = jnp.zeros((LANES,), jnp.int32)
    fzeros = jnp.zeros((LANES,), jnp.float32)
    lane = lax.iota(jnp.int32, LANES)

    with jax.named_scope("zerofill"):
        def _zero_zbuf(r, c):
            for cc in range(C // LANES):
                zbuf[r, pl.ds(cc * LANES, LANES)] = fzeros
            return c

        lax.fori_loop(0, CH, _zero_zbuf, 0)

        def _zero_arr(i, c):
            arr_v[pl.ds(i * LANES, LANES)] = izeros
            return c

        lax.fori_loop(0, M // LANES, _zero_arr, 0)

    def _pass1(j, run):
        d = tgt_v[pl.ds(j * LANES, LANES)]
        c = plsc.cumsum(d) + run
        start = c - d
        jid = lane + j * LANES
        m = (d > 0) & (start < M)
        plsc.store_scatter(arr_v, [start], jid, mask=m)
        return c[15]

    with jax.named_scope("pass1"):
        valid = lax.fori_loop(0, L // LANES, _pass1, jnp.int32(0))

    kd = jnp.clip((valid + CH - 1) // CH, 0, M // CH)
    nd = jnp.maximum((kd - par + 1) // 2, 0)
    nz = CPW - nd

    with jax.named_scope("firezeros"):
        def _fire_zero(i, c):
            k = 2 * i + par
            pltpu.async_copy(zbuf, out_hbm.at[pl.ds(b * M + k * CH, CH)],
                             zwsem)
            return c

        lax.fori_loop(nd, CPW, _fire_zero, 0)

    def _pass2(i, carry):
        a = arr_v[pl.ds(i * LANES, LANES)]
        mval = jnp.maximum(plsc.cummax(a), carry)
        row = b * L + jnp.minimum(mval, L - 1)
        gidx_v[pl.ds(i * LANES, LANES)] = row
        return mval[15]

    with jax.named_scope("pass2"):
        lax.fori_loop(0, kd * (CH // LANES), _pass2, jnp.int32(0))

    bufs = (buf_a, buf_b)
    gsems = (gsem_a, gsem_b)
    wsems = (wsem_a, wsem_b)

    def _chunk(i):
        return 2 * i + par

    def _start_gather(i, p):
        pltpu.async_copy(
            x_hbm.at[gidx_v.at[pl.ds(_chunk(i) * CH, CH)]], bufs[p], gsems[p])

    def _wait_gather(i, p):
        pltpu.make_async_copy(
            x_hbm.at[gidx_v.at[pl.ds(_chunk(i) * CH, CH)]], bufs[p],
            gsems[p]).wait()

    def _drain_write(sem):
        pltpu.make_async_copy(zbuf, out_hbm.at[pl.ds(b * M, CH)], sem).wait()

    _chunk_scope = jax.named_scope("chunkloop")
    _chunk_scope.__enter__()
    for i in range(CPW):
        p = i % 2

        if i == 0:
            @pl.when(nd > 0)
            def _prime():
                _start_gather(0, 0)

        @pl.when(i < nd)
        def _data_chunk():
            _wait_gather(i, p)
            vs = valid - _chunk(i) * CH

            def _zero_tail(r, c):
                for cc in range(C // LANES):
                    bufs[p][r, pl.ds(cc * LANES, LANES)] = fzeros
                return c

            lax.fori_loop(jnp.clip(vs, 0, CH), CH, _zero_tail, 0)
            pltpu.async_copy(
                bufs[p], out_hbm.at[pl.ds(b * M + _chunk(i) * CH, CH)],
                wsems[p])

        if i + 1 < CPW:
            q = (i + 1) % 2

            @pl.when(i + 1 < nd)
            def _next_gather():
                if i >= 1:
                    _drain_write(wsems[q])
                _start_gather(i + 1, q)

    @pl.when(nd >= 1)
    def _drain_a():
        _drain_write(wsem_a)

    @pl.when(nd >= 2)
    def _drain_b():
        _drain_write(wsem_b)

    def _drain_zero(i, c):
        pltpu.make_async_copy(
            zbuf, out_hbm.at[pl.ds(b * M, CH)], zwsem).wait()
        return c

    lax.fori_loop(0, nz, _drain_zero, 0)
    _chunk_scope.__exit__(None, None, None)


def _sc_expand(x2d, tgt):
    mesh = plsc.VectorSubcoreMesh(core_axis_name="c", subcore_axis_name="s")
    kern = pl.kernel(
        _sc_body,
        out_type=jax.ShapeDtypeStruct((B * M, C), jnp.float32),
        mesh=mesh,
        scratch_types=[
            pltpu.VMEM((L,), jnp.int32),
            pltpu.VMEM((M,), jnp.int32),
            pltpu.VMEM((M,), jnp.int32),
            pltpu.VMEM((CH, C), jnp.float32),
            pltpu.VMEM((CH, C), jnp.float32),
            pltpu.VMEM((CH, C), jnp.float32),
            pltpu.SemaphoreType.DMA,
            pltpu.SemaphoreType.DMA,
            pltpu.SemaphoreType.DMA,
            pltpu.SemaphoreType.DMA,
            pltpu.SemaphoreType.DMA,
        ],
        compiler_params=pltpu.CompilerParams(needs_layout_passes=False),
    )
    return kern(x2d, tgt)


def _layernorm(h, g, bb):
    mu = jnp.mean(h, axis=-1, keepdims=True)
    var = jnp.mean((h - mu) ** 2, axis=-1, keepdims=True)
    return (h - mu) * lax.rsqrt(var + 1e-5) * g + bb


def _conv_block(X, w0, w1, w2, bias):
    z = jnp.zeros((1, C), jnp.float32)
    Xm = jnp.concatenate([z, X[:-1]], axis=0)
    Xp = jnp.concatenate([X[1:], z], axis=0)
    f32 = jnp.float32
    h = (jnp.dot(Xm, w0, preferred_element_type=f32)
         + jnp.dot(X, w1, preferred_element_type=f32)
         + jnp.dot(Xp, w2, preferred_element_type=f32))
    return h + bias


def _vp_body(x_ref, w10, w11, w12, b1, g1, e1, w20, w21, w22, b2, g2, e2,
             lw, lb, out_ref):
    X = x_ref[0]
    h = _conv_block(X, w10[:], w11[:], w12[:], b1[:])
    h = _layernorm(jnp.maximum(h, 0.0), g1[:], e1[:])
    h = _conv_block(h, w20[:], w21[:], w22[:], b2[:])
    h = _layernorm(jnp.maximum(h, 0.0), g2[:], e2[:])
    o = jnp.dot(h, lw[:], preferred_element_type=jnp.float32) + lb[:]
    out_ref[0] = jnp.maximum(o, 0.0)


def _variance_predictor(x, conv1_w, conv1_b, ln1_g, ln1_b, conv2_w, conv2_b,
                        ln2_g, ln2_b, lin_w, lin_b):
    full2d = pl.BlockSpec((C, C), lambda i: (0, 0))
    vec = pl.BlockSpec((C,), lambda i: (0,))
    out = pl.pallas_call(
        _vp_body,
        grid=(B,),
        in_specs=[
            pl.BlockSpec((1, L, C), lambda i: (i, 0, 0)),
            full2d, full2d, full2d, vec, vec, vec,
            full2d, full2d, full2d, vec, vec, vec,
            pl.BlockSpec((C, 1), lambda i: (0, 0)),
            pl.BlockSpec((1,), lambda i: (0,)),
        ],
        out_specs=pl.BlockSpec((1, L, 1), lambda i: (i, 0, 0)),
        out_shape=jax.ShapeDtypeStruct((B, L, 1), jnp.float32),
    )(
        x,
        conv1_w[:, :, 0].T, conv1_w[:, :, 1].T, conv1_w[:, :, 2].T,
        conv1_b, ln1_g, ln1_b,
        conv2_w[:, :, 0].T, conv2_w[:, :, 1].T, conv2_w[:, :, 2].T,
        conv2_b, ln2_g, ln2_b,
        lin_w.T, lin_b,
    )
    return out.reshape(B, L)


def kernel(x, target, mel_max_length, conv1_w, conv1_b, ln1_g, ln1_b,
           conv2_w, conv2_b, ln2_g, ln2_b, lin_w, lin_b):
    ldp = _variance_predictor(x, conv1_w, conv1_b, ln1_g, ln1_b,
                              conv2_w, conv2_b, ln2_g, ln2_b, lin_w, lin_b)
    out = _sc_expand(x.reshape(B * L, C), target).reshape(B, M, C)
    return (out, ldp)
